# Initial kernel scaffold; baseline (speedup 1.0000x reference)
import jax
import jax.numpy as jnp
from jax.experimental import pallas as pl


def kernel(x, batch, query, Wk, bk, Wv, bv):
    raise NotImplementedError("write your pallas kernel here")


# speedup vs baseline: 5.4942x; 5.4942x over previous
import jax
import jax.numpy as jnp
from jax import lax
from jax.experimental import pallas as pl
from jax.experimental.pallas import tpu as pltpu
from jax.experimental.pallas import tpu_sc as plsc

N = 100000
D = 128
DO = 128
B = 512
L = 16
NW = 32
CH = 128
NCH = (N + CH - 1) // CH
DV = D // L
RB = 2000
NRB = N // RB


def _lane_shuffle(v, idx):
    return lax.gather(
        v, idx[:, None],
        dimension_numbers=lax.GatherDimensionNumbers(
            offset_dims=(), collapsed_slice_dims=(0,), start_index_map=(0,)),
        slice_sizes=(1,),
        mode=lax.GatherScatterMode.PROMISE_IN_BOUNDS)


def _tc_scores(x_ref, wk_ref, bk_ref, q_ref, e_ref):
    k = lax.dot_general(x_ref[...], wk_ref[...],
                        (((1,), (1,)), ((), ())))
    k = k + bk_ref[...]
    s = lax.dot_general(k, q_ref[...], (((1,), (0,)), ((), ())))
    e_ref[...] = jnp.exp(s[:, 0])[None, None, :]


def _sc_body(x_hbm, b_hbm, e_hbm, outacc_hbm, outden_hbm,
             xbuf, bbuf, ebuf, acc, dacc):
    wid = lax.axis_index("c") * 16 + lax.axis_index("s")

    iota16 = lax.iota(jnp.int32, L)
    unit0 = jnp.where(iota16 == 0, jnp.float32(1.0), jnp.float32(0.0))
    zv = jnp.zeros((L,), jnp.float32)

    def zero_step(i, _):
        for j in range(DV):
            acc[pl.ds(i * D + L * j, L)] = zv
        dacc[pl.ds(i * L, L)] = zv
        return 0

    lax.fori_loop(0, B, zero_step, 0)

    nch_w = (NCH - wid + NW - 1) // NW

    def chunk_step(ci, _):
        c = wid + ci * NW
        start = jnp.minimum(c * CH, N - CH)
        pltpu.sync_copy(x_hbm.at[pl.ds(start * D, CH * D)], xbuf)
        pltpu.sync_copy(b_hbm.at[pl.ds(start, CH)], bbuf)
        pltpu.sync_copy(e_hbm.at[pl.ds(start, CH)], ebuf)

        def row_step(ri, _):
            segv = bbuf[pl.ds(ri * L, L)]
            e16 = ebuf[pl.ds(ri * L, L)]
            for u in range(L):
                r = ri * L + u
                base = r * D
                validf = ((start + r) >= (c * CH)).astype(jnp.float32)
                ev = _lane_shuffle(e16, iota16 * 0 + u) * validf
                seg = segv[u]
                abase = seg * D
                for j in range(DV):
                    plsc.addupdate(acc.at[pl.ds(abase + L * j, L)],
                                   ev * xbuf[pl.ds(base + L * j, L)])
                plsc.addupdate(dacc.at[pl.ds(seg * L, L)], ev * unit0)
            return 0

        lax.fori_loop(0, CH // L, row_step, 0)
        return 0

    lax.fori_loop(0, nch_w, chunk_step, 0)

    pltpu.sync_copy(acc, outacc_hbm.at[wid])
    pltpu.sync_copy(dacc, outden_hbm.at[wid])


def _tc_finalize(acc_ref, den_ref, wv_ref, bv_ref, o_ref):
    esum = jnp.sum(acc_ref[...], axis=0)
    den = jnp.sum(den_ref[...], axis=(0, 2))
    safe = den + jnp.float32(1e-16)
    pooled = esum / safe[:, None]
    out = lax.dot_general(pooled, wv_ref[...],
                          (((1,), (1,)), ((), ())),
                          preferred_element_type=jnp.float32)
    o_ref[...] = out + (den / safe)[:, None] * bv_ref[...]


@jax.jit
def kernel(x, batch, query, Wk, bk, Wv, bv):
    x_flat = x.reshape(-1)
    batchi = batch.astype(jnp.int32)

    e2d = pl.pallas_call(
        _tc_scores,
        grid=(NRB,),
        in_specs=[
            pl.BlockSpec((RB, D), lambda i: (i, 0)),
            pl.BlockSpec((D, D), lambda i: (0, 0)),
            pl.BlockSpec((1, D), lambda i: (0, 0)),
            pl.BlockSpec((D, 1), lambda i: (0, 0)),
        ],
        out_specs=pl.BlockSpec((1, 1, RB), lambda i: (i, 0, 0)),
        out_shape=jax.ShapeDtypeStruct((NRB, 1, RB), jnp.float32),
    )(x, Wk, bk.reshape(1, D), query.reshape(D, 1))
    e = e2d.reshape(N)

    mesh = plsc.VectorSubcoreMesh(core_axis_name="c", subcore_axis_name="s")
    sc = pl.kernel(
        _sc_body,
        out_type=[jax.ShapeDtypeStruct((NW, B * D), jnp.float32),
                  jax.ShapeDtypeStruct((NW, B * L), jnp.float32)],
        mesh=mesh,
        scratch_types=[
            pltpu.VMEM((CH * D,), jnp.float32),
            pltpu.VMEM((CH,), jnp.int32),
            pltpu.VMEM((CH,), jnp.float32),
            pltpu.VMEM((B * D,), jnp.float32),
            pltpu.VMEM((B * L,), jnp.float32),
        ],
    )
    outacc, outden = sc(x_flat, batchi, e)

    accs = outacc.reshape(NW, B, D)
    dens = outden.reshape(NW, B, L)

    out = pl.pallas_call(
        _tc_finalize,
        out_shape=jax.ShapeDtypeStruct((B, DO), jnp.float32),
    )(accs, dens, Wv, bv.reshape(1, DO))
    return out


# speedup vs baseline: 6.6284x; 1.2064x over previous
import jax
import jax.numpy as jnp
from jax import lax
from jax.experimental import pallas as pl
from jax.experimental.pallas import tpu as pltpu
from jax.experimental.pallas import tpu_sc as plsc

N = 100000
D = 128
DO = 128
B = 512
L = 16
NW = 32
CH = 128
NCH = (N + CH - 1) // CH
DV = D // L
RB = 2000
NRB = N // RB


def _lane_shuffle(v, idx):
    return lax.gather(
        v, idx[:, None],
        dimension_numbers=lax.GatherDimensionNumbers(
            offset_dims=(), collapsed_slice_dims=(0,), start_index_map=(0,)),
        slice_sizes=(1,),
        mode=lax.GatherScatterMode.PROMISE_IN_BOUNDS)


def _tc_scores(x_ref, wk_ref, bk_ref, q_ref, e_ref):
    k = lax.dot_general(x_ref[...], wk_ref[...],
                        (((1,), (1,)), ((), ())))
    k = k + bk_ref[...]
    s = lax.dot_general(k, q_ref[...], (((1,), (0,)), ((), ())))
    e_ref[...] = jnp.exp(s[:, 0])[None, None, :]


NSL = 26


def _sc_body(x_hbm, b_hbm, e_hbm, outacc_hbm, outden_hbm,
             xb0, xb1, bb0, bb1, eb0, eb1, acc, dacc,
             sx0, sx1, sb0, sb1, se0, se1):
    wid = lax.axis_index("c") * 16 + lax.axis_index("s")
    bufs = ((xb0, bb0, eb0, sx0, sb0, se0), (xb1, bb1, eb1, sx1, sb1, se1))

    iota16 = lax.iota(jnp.int32, L)
    unit0 = jnp.where(iota16 == 0, jnp.float32(1.0), jnp.float32(0.0))
    zv = jnp.zeros((L,), jnp.float32)

    def chunk_info(slot):
        cid = wid + slot * NW
        c_eff = jnp.minimum(cid, NCH - 1)
        start = jnp.minimum(c_eff * CH, N - CH)
        return cid, c_eff, start

    def dma_descs(slot, bufset):
        xb, bb, eb, sx, sb, se = bufset
        _, _, start = chunk_info(slot)
        return (pltpu.make_async_copy(x_hbm.at[pl.ds(start * D, CH * D)],
                                      xb, sx),
                pltpu.make_async_copy(b_hbm.at[pl.ds(start, CH)], bb, sb),
                pltpu.make_async_copy(e_hbm.at[pl.ds(start, CH)], eb, se))

    for d in dma_descs(jnp.int32(0), bufs[0]):
        d.start()
    for d in dma_descs(jnp.int32(1), bufs[1]):
        d.start()

    def zero_step(i, _):
        for j in range(DV):
            acc[pl.ds(i * D + L * j, L)] = zv
        dacc[pl.ds(i * L, L)] = zv
        return 0

    lax.fori_loop(0, B, zero_step, 0)

    def process(slot, bufset):
        xb, bb, eb, _, _, _ = bufset
        cid, c_eff, start = chunk_info(slot)
        okf = (cid < NCH).astype(jnp.float32)

        def row_step(ri, _):
            segv = bb[pl.ds(ri * L, L)]
            e16 = eb[pl.ds(ri * L, L)]
            for u in range(L):
                r = ri * L + u
                base = r * D
                validf = ((start + r) >= (c_eff * CH)).astype(jnp.float32)
                ev = _lane_shuffle(e16, iota16 * 0 + u) * (validf * okf)
                seg = segv[u]
                abase = seg * D
                for j in range(DV):
                    plsc.addupdate(acc.at[pl.ds(abase + L * j, L)],
                                   ev * xb[pl.ds(base + L * j, L)])
                plsc.addupdate(dacc.at[pl.ds(seg * L, L)], ev * unit0)
            return 0

        lax.fori_loop(0, CH // L, row_step, 0)

    def pair_step(i, _):
        t = i * 2
        for b in range(2):
            slot = t + b
            for d in dma_descs(slot, bufs[b]):
                d.wait()
            process(slot, bufs[b])
            nxt = slot + 2

            @pl.when(nxt < NSL)
            def _():
                for d in dma_descs(nxt, bufs[b]):
                    d.start()
        return 0

    lax.fori_loop(0, NSL // 2, pair_step, 0)

    pltpu.sync_copy(acc, outacc_hbm.at[wid])
    pltpu.sync_copy(dacc, outden_hbm.at[wid])


def _tc_finalize(acc_ref, den_ref, wv_ref, bv_ref, o_ref):
    esum = jnp.sum(acc_ref[...], axis=0)
    den = jnp.sum(den_ref[...], axis=(0, 2))
    safe = den + jnp.float32(1e-16)
    pooled = esum / safe[:, None]
    out = lax.dot_general(pooled, wv_ref[...],
                          (((1,), (1,)), ((), ())),
                          preferred_element_type=jnp.float32)
    o_ref[...] = out + (den / safe)[:, None] * bv_ref[...]


@jax.jit
def kernel(x, batch, query, Wk, bk, Wv, bv):
    x_flat = x.reshape(-1)
    batchi = batch.astype(jnp.int32)

    e2d = pl.pallas_call(
        _tc_scores,
        grid=(NRB,),
        in_specs=[
            pl.BlockSpec((RB, D), lambda i: (i, 0)),
            pl.BlockSpec((D, D), lambda i: (0, 0)),
            pl.BlockSpec((1, D), lambda i: (0, 0)),
            pl.BlockSpec((D, 1), lambda i: (0, 0)),
        ],
        out_specs=pl.BlockSpec((1, 1, RB), lambda i: (i, 0, 0)),
        out_shape=jax.ShapeDtypeStruct((NRB, 1, RB), jnp.float32),
    )(x, Wk, bk.reshape(1, D), query.reshape(D, 1))
    e = e2d.reshape(N)

    mesh = plsc.VectorSubcoreMesh(core_axis_name="c", subcore_axis_name="s")
    sc = pl.kernel(
        _sc_body,
        out_type=[jax.ShapeDtypeStruct((NW, B * D), jnp.float32),
                  jax.ShapeDtypeStruct((NW, B * L), jnp.float32)],
        mesh=mesh,
        scratch_types=[
            pltpu.VMEM((CH * D,), jnp.float32),
            pltpu.VMEM((CH * D,), jnp.float32),
            pltpu.VMEM((CH,), jnp.int32),
            pltpu.VMEM((CH,), jnp.int32),
            pltpu.VMEM((CH,), jnp.float32),
            pltpu.VMEM((CH,), jnp.float32),
            pltpu.VMEM((B * D,), jnp.float32),
            pltpu.VMEM((B * L,), jnp.float32),
            pltpu.SemaphoreType.DMA,
            pltpu.SemaphoreType.DMA,
            pltpu.SemaphoreType.DMA,
            pltpu.SemaphoreType.DMA,
            pltpu.SemaphoreType.DMA,
            pltpu.SemaphoreType.DMA,
        ],
    )
    outacc, outden = sc(x_flat, batchi, e)

    accs = outacc.reshape(NW, B, D)
    dens = outden.reshape(NW, B, L)

    out = pl.pallas_call(
        _tc_finalize,
        out_shape=jax.ShapeDtypeStruct((B, DO), jnp.float32),
    )(accs, dens, Wv, bv.reshape(1, DO))
    return out


# speedup vs baseline: 6.7132x; 1.0128x over previous
import jax
import jax.numpy as jnp
from jax import lax
from jax.experimental import pallas as pl
from jax.experimental.pallas import tpu as pltpu
from jax.experimental.pallas import tpu_sc as plsc

N = 100000
D = 128
DO = 128
B = 512
L = 16
NW = 32
CH = 128
NCH = (N + CH - 1) // CH
DV = D // L
RB = 2000
NRB = N // RB


def _lane_shuffle(v, idx):
    return lax.gather(
        v, idx[:, None],
        dimension_numbers=lax.GatherDimensionNumbers(
            offset_dims=(), collapsed_slice_dims=(0,), start_index_map=(0,)),
        slice_sizes=(1,),
        mode=lax.GatherScatterMode.PROMISE_IN_BOUNDS)


def _tc_scores(x_ref, wk_ref, bk_ref, q_ref, e_ref):
    k = lax.dot_general(x_ref[...], wk_ref[...],
                        (((1,), (1,)), ((), ())))
    k = k + bk_ref[...]
    s = lax.dot_general(k, q_ref[...], (((1,), (0,)), ((), ())))
    e_ref[...] = jnp.exp(s[:, 0])[None, None, :]


NSL = 26


def _sc_body(x_hbm, b_hbm, e_hbm, outacc_hbm, outden_hbm,
             xb0, xb1, bb0, bb1, eb0, eb1, acc, dacc,
             sx0, sx1, sb0, sb1, se0, se1):
    wid = lax.axis_index("c") * 16 + lax.axis_index("s")
    bufs = ((xb0, bb0, eb0, sx0, sb0, se0), (xb1, bb1, eb1, sx1, sb1, se1))

    iota16 = lax.iota(jnp.int32, L)
    unit0 = jnp.where(iota16 == 0, jnp.float32(1.0), jnp.float32(0.0))
    zv = jnp.zeros((L,), jnp.float32)

    def chunk_info(slot):
        cid = wid + slot * NW
        c_eff = jnp.minimum(cid, NCH - 1)
        start = jnp.minimum(c_eff * CH, N - CH)
        return cid, c_eff, start

    def dma_descs(slot, bufset):
        xb, bb, eb, sx, sb, se = bufset
        _, _, start = chunk_info(slot)
        return (pltpu.make_async_copy(x_hbm.at[pl.ds(start, CH), :], xb, sx),
                pltpu.make_async_copy(b_hbm.at[pl.ds(start, CH)], bb, sb),
                pltpu.make_async_copy(e_hbm.at[pl.ds(start, CH)], eb, se))

    for d in dma_descs(jnp.int32(0), bufs[0]):
        d.start()
    for d in dma_descs(jnp.int32(1), bufs[1]):
        d.start()

    def zero_step(i, _):
        for j in range(DV):
            acc[i, pl.ds(L * j, L)] = zv
        dacc[pl.ds(i * L, L)] = zv
        return 0

    lax.fori_loop(0, B, zero_step, 0)

    def process(slot, bufset):
        xb, bb, eb, _, _, _ = bufset
        cid, c_eff, start = chunk_info(slot)
        okf = (cid < NCH).astype(jnp.float32)

        def row_step(ri, _):
            segv = bb[pl.ds(ri * L, L)]
            e16 = eb[pl.ds(ri * L, L)]
            vmask = jnp.where((iota16 + (start + ri * L)) >= (c_eff * CH),
                              okf, jnp.float32(0.0))
            e16m = e16 * vmask
            for u in range(L):
                r = ri * L + u
                ev = _lane_shuffle(e16m, iota16 * 0 + u)
                seg = segv[u]
                for j in range(DV):
                    plsc.addupdate(acc.at[seg, pl.ds(L * j, L)],
                                   ev * xb[r, pl.ds(L * j, L)])
                plsc.addupdate(dacc.at[pl.ds(seg * L, L)], ev * unit0)
            return 0

        lax.fori_loop(0, CH // L, row_step, 0)

    def pair_step(i, _):
        t = i * 2
        for b in range(2):
            slot = t + b
            for d in dma_descs(slot, bufs[b]):
                d.wait()
            process(slot, bufs[b])
            nxt = slot + 2

            @pl.when(nxt < NSL)
            def _():
                for d in dma_descs(nxt, bufs[b]):
                    d.start()
        return 0

    lax.fori_loop(0, NSL // 2, pair_step, 0)

    pltpu.sync_copy(acc, outacc_hbm.at[wid, :, :])
    pltpu.sync_copy(dacc, outden_hbm.at[wid])


def _tc_finalize(acc_ref, den_ref, wv_ref, bv_ref, o_ref):
    esum = jnp.sum(acc_ref[...], axis=0)
    den = jnp.sum(den_ref[...], axis=(0, 2))
    safe = den + jnp.float32(1e-16)
    pooled = esum / safe[:, None]
    out = lax.dot_general(pooled, wv_ref[...],
                          (((1,), (1,)), ((), ())),
                          preferred_element_type=jnp.float32)
    o_ref[...] = out + (den / safe)[:, None] * bv_ref[...]


@jax.jit
def kernel(x, batch, query, Wk, bk, Wv, bv):
    batchi = batch.astype(jnp.int32)

    e2d = pl.pallas_call(
        _tc_scores,
        grid=(NRB,),
        in_specs=[
            pl.BlockSpec((RB, D), lambda i: (i, 0)),
            pl.BlockSpec((D, D), lambda i: (0, 0)),
            pl.BlockSpec((1, D), lambda i: (0, 0)),
            pl.BlockSpec((D, 1), lambda i: (0, 0)),
        ],
        out_specs=pl.BlockSpec((1, 1, RB), lambda i: (i, 0, 0)),
        out_shape=jax.ShapeDtypeStruct((NRB, 1, RB), jnp.float32),
    )(x, Wk, bk.reshape(1, D), query.reshape(D, 1))
    e = e2d.reshape(N)

    mesh = plsc.VectorSubcoreMesh(core_axis_name="c", subcore_axis_name="s")
    sc = pl.kernel(
        _sc_body,
        out_type=[jax.ShapeDtypeStruct((NW, B, D), jnp.float32),
                  jax.ShapeDtypeStruct((NW, B * L), jnp.float32)],
        mesh=mesh,
        scratch_types=[
            pltpu.VMEM((CH, D), jnp.float32),
            pltpu.VMEM((CH, D), jnp.float32),
            pltpu.VMEM((CH,), jnp.int32),
            pltpu.VMEM((CH,), jnp.int32),
            pltpu.VMEM((CH,), jnp.float32),
            pltpu.VMEM((CH,), jnp.float32),
            pltpu.VMEM((B, D), jnp.float32),
            pltpu.VMEM((B * L,), jnp.float32),
            pltpu.SemaphoreType.DMA,
            pltpu.SemaphoreType.DMA,
            pltpu.SemaphoreType.DMA,
            pltpu.SemaphoreType.DMA,
            pltpu.SemaphoreType.DMA,
            pltpu.SemaphoreType.DMA,
        ],
    )
    accs, dens2 = sc(x, batchi, e)
    dens = dens2.reshape(NW, B, L)

    out = pl.pallas_call(
        _tc_finalize,
        out_shape=jax.ShapeDtypeStruct((B, DO), jnp.float32),
    )(accs, dens, Wv, bv.reshape(1, DO))
    return out


# speedup vs baseline: 10.6520x; 1.5867x over previous
import jax
import jax.numpy as jnp
from jax import lax
from jax.experimental import pallas as pl
from jax.experimental.pallas import tpu as pltpu
from jax.experimental.pallas import tpu_sc as plsc

N = 100000
D = 128
DO = 128
B = 512
L = 16
NW = 32
CH = 128
NCH = (N + CH - 1) // CH
DV = D // L
RB = 2000
NRB = N // RB


def _lane_shuffle(v, idx):
    return lax.gather(
        v, idx[:, None],
        dimension_numbers=lax.GatherDimensionNumbers(
            offset_dims=(), collapsed_slice_dims=(0,), start_index_map=(0,)),
        slice_sizes=(1,),
        mode=lax.GatherScatterMode.PROMISE_IN_BOUNDS)


def _tc_scores(x_ref, wk_ref, bk_ref, q_ref, e_ref):
    k = lax.dot_general(x_ref[...], wk_ref[...],
                        (((1,), (1,)), ((), ())))
    k = k + bk_ref[...]
    s = lax.dot_general(k, q_ref[...], (((1,), (0,)), ((), ())))
    e_ref[...] = jnp.exp(s[:, 0])[None, None, :]


NSL = 26


def _sc_body(x_hbm, b_hbm, e_hbm, outacc_hbm, outden_hbm,
             xb0, xb1, bb0, bb1, eb0, eb1, acc, dacc,
             sx0, sx1, sb0, sb1, se0, se1):
    wid = lax.axis_index("c") * 16 + lax.axis_index("s")
    bufs = ((xb0, bb0, eb0, sx0, sb0, se0), (xb1, bb1, eb1, sx1, sb1, se1))

    iota16 = lax.iota(jnp.int32, L)
    unit0 = jnp.where(iota16 == 0, jnp.float32(1.0), jnp.float32(0.0))
    zv = jnp.zeros((L,), jnp.float32)

    def chunk_info(slot):
        cid = wid + slot * NW
        c_eff = jnp.minimum(cid, NCH - 1)
        start = jnp.minimum(c_eff * CH, N - CH)
        return cid, c_eff, start

    def dma_descs(slot, bufset):
        xb, bb, eb, sx, sb, se = bufset
        _, _, start = chunk_info(slot)
        return (pltpu.make_async_copy(x_hbm.at[pl.ds(start, CH), :], xb, sx),
                pltpu.make_async_copy(b_hbm.at[pl.ds(start, CH)], bb, sb),
                pltpu.make_async_copy(e_hbm.at[pl.ds(start, CH)], eb, se))

    for d in dma_descs(jnp.int32(0), bufs[0]):
        d.start()
    for d in dma_descs(jnp.int32(1), bufs[1]):
        d.start()

    def zero_step(i, _):
        for j in range(DV):
            acc[i, pl.ds(L * j, L)] = zv
        dacc[pl.ds(i * L, L)] = zv
        return 0

    lax.fori_loop(0, B, zero_step, 0)

    def process(slot, bufset):
        xb, bb, eb, _, _, _ = bufset
        cid, c_eff, start = chunk_info(slot)
        okf = (cid < NCH).astype(jnp.float32)

        def row_step(ri, _):
            segv = bb[pl.ds(ri * L, L)]
            e16 = eb[pl.ds(ri * L, L)]
            vmask = jnp.where((iota16 + (start + ri * L)) >= (c_eff * CH),
                              okf, jnp.float32(0.0))
            e16m = e16 * vmask
            s0 = segv[0]
            uniform = s0 == segv[L - 1]

            @pl.when(uniform)
            def _():
                regs = [jnp.zeros((L,), jnp.float32) for _ in range(DV)]
                for u in range(L):
                    ev = _lane_shuffle(e16m, iota16 * 0 + u)
                    for j in range(DV):
                        regs[j] = regs[j] + ev * xb[ri * L + u,
                                                    pl.ds(L * j, L)]
                for j in range(DV):
                    plsc.addupdate(acc.at[s0, pl.ds(L * j, L)], regs[j])
                dsum = e16m
                for k in (8, 4, 2, 1):
                    dsum = dsum + _lane_shuffle(dsum, iota16 ^ k)
                plsc.addupdate(dacc.at[pl.ds(s0 * L, L)], dsum * unit0)

            @pl.when(jnp.logical_not(uniform))
            def _():
                for u in range(L):
                    r = ri * L + u
                    ev = _lane_shuffle(e16m, iota16 * 0 + u)
                    seg = segv[u]
                    for j in range(DV):
                        plsc.addupdate(acc.at[seg, pl.ds(L * j, L)],
                                       ev * xb[r, pl.ds(L * j, L)])
                    plsc.addupdate(dacc.at[pl.ds(seg * L, L)], ev * unit0)
            return 0

        lax.fori_loop(0, CH // L, row_step, 0)

    def pair_step(i, _):
        t = i * 2
        for b in range(2):
            slot = t + b
            for d in dma_descs(slot, bufs[b]):
                d.wait()
            process(slot, bufs[b])
            nxt = slot + 2

            @pl.when(nxt < NSL)
            def _():
                for d in dma_descs(nxt, bufs[b]):
                    d.start()
        return 0

    lax.fori_loop(0, NSL // 2, pair_step, 0)

    pltpu.sync_copy(acc, outacc_hbm.at[wid, :, :])
    pltpu.sync_copy(dacc, outden_hbm.at[wid])


def _tc_finalize(acc_ref, den_ref, wv_ref, bv_ref, o_ref):
    esum = jnp.sum(acc_ref[...], axis=0)
    den = jnp.sum(den_ref[...], axis=(0, 2))
    safe = den + jnp.float32(1e-16)
    pooled = esum / safe[:, None]
    out = lax.dot_general(pooled, wv_ref[...],
                          (((1,), (1,)), ((), ())),
                          preferred_element_type=jnp.float32)
    o_ref[...] = out + (den / safe)[:, None] * bv_ref[...]


@jax.jit
def kernel(x, batch, query, Wk, bk, Wv, bv):
    batchi = batch.astype(jnp.int32)

    e2d = pl.pallas_call(
        _tc_scores,
        grid=(NRB,),
        in_specs=[
            pl.BlockSpec((RB, D), lambda i: (i, 0)),
            pl.BlockSpec((D, D), lambda i: (0, 0)),
            pl.BlockSpec((1, D), lambda i: (0, 0)),
            pl.BlockSpec((D, 1), lambda i: (0, 0)),
        ],
        out_specs=pl.BlockSpec((1, 1, RB), lambda i: (i, 0, 0)),
        out_shape=jax.ShapeDtypeStruct((NRB, 1, RB), jnp.float32),
    )(x, Wk, bk.reshape(1, D), query.reshape(D, 1))
    e = e2d.reshape(N)

    mesh = plsc.VectorSubcoreMesh(core_axis_name="c", subcore_axis_name="s")
    sc = pl.kernel(
        _sc_body,
        out_type=[jax.ShapeDtypeStruct((NW, B, D), jnp.float32),
                  jax.ShapeDtypeStruct((NW, B * L), jnp.float32)],
        mesh=mesh,
        scratch_types=[
            pltpu.VMEM((CH, D), jnp.float32),
            pltpu.VMEM((CH, D), jnp.float32),
            pltpu.VMEM((CH,), jnp.int32),
            pltpu.VMEM((CH,), jnp.int32),
            pltpu.VMEM((CH,), jnp.float32),
            pltpu.VMEM((CH,), jnp.float32),
            pltpu.VMEM((B, D), jnp.float32),
            pltpu.VMEM((B * L,), jnp.float32),
            pltpu.SemaphoreType.DMA,
            pltpu.SemaphoreType.DMA,
            pltpu.SemaphoreType.DMA,
            pltpu.SemaphoreType.DMA,
            pltpu.SemaphoreType.DMA,
            pltpu.SemaphoreType.DMA,
        ],
    )
    accs, dens2 = sc(x, batchi, e)
    dens = dens2.reshape(NW, B, L)

    out = pl.pallas_call(
        _tc_finalize,
        out_shape=jax.ShapeDtypeStruct((B, DO), jnp.float32),
    )(accs, dens, Wv, bv.reshape(1, DO))
    return out
